# Initial kernel scaffold; baseline (speedup 1.0000x reference)
import jax
import jax.numpy as jnp
from jax.experimental import pallas as pl


def kernel(X_tilde1, adj1, W_enc1, b_enc1, W_enc2, b_enc2, W_enc3, b_enc3, W_dec1, b_dec1, W_dec2, b_dec2, W_dec3, b_dec3, W_xbar, b_xbar, Wg1, Wg2, Wg3, Wg4, Wg5, Wg6):
    raise NotImplementedError("write your pallas kernel here")


# speedup vs baseline: 1.0029x; 1.0029x over previous
import functools

import jax
import jax.numpy as jnp
from jax.experimental import pallas as pl

_N = 4096
_SIGMA = 0.5
_BM = 256
_BS = 512

_f32 = jnp.float32
_bf16 = jnp.bfloat16


def _dot(a, b):
    return jnp.dot(a.astype(_bf16), b.astype(_bf16),
                   preferred_element_type=_f32)


def _k1_body(x_ref, we1, be1, we2, be2, we3, be3,
             wd1, bd1, wd2, bd2, wd3, bd3, wxb, bxb, wg1,
             z1_ref, z2_ref, z3_ref, y1_ref, xhat_ref):
    x = x_ref[...]
    z1 = jax.nn.relu(_dot(x, we1[...]) + be1[...])
    z2 = jax.nn.relu(_dot(z1, we2[...]) + be2[...])
    z3 = _dot(z2, we3[...]) + be3[...]
    z1_ref[...] = z1
    z2_ref[...] = z2
    z3_ref[...] = z3
    y1_ref[...] = _dot(x, wg1[...])
    d = jax.nn.relu(_dot(z3, wd1[...]) + bd1[...])
    d = jax.nn.relu(_dot(d, wd2[...]) + bd2[...])
    d = jax.nn.relu(_dot(d, wd3[...]) + bd3[...])
    xhat_ref[...] = _dot(d, wxb[...]) + bxb[...]


def _full(a):
    nd = a.ndim
    return pl.BlockSpec(a.shape, lambda i: (0,) * nd)


def _ae_chain(x, we1, be1, we2, be2, we3, be3,
              wd1, bd1, wd2, bd2, wd3, bd3, wxb, bxb, wg1_bf):
    n = x.shape[0]
    grid = (n // _BM,)
    row = lambda k: pl.BlockSpec((_BM, k), lambda i: (i, 0))
    consts = [we1, be1, we2, be2, we3, be3,
              wd1, bd1, wd2, bd2, wd3, bd3, wxb, bxb, wg1_bf]
    return pl.pallas_call(
        _k1_body,
        grid=grid,
        in_specs=[row(512)] + [_full(c) for c in consts],
        out_specs=[row(128), row(64), row(20), row(128), row(512)],
        out_shape=[
            jax.ShapeDtypeStruct((n, 128), _f32),
            jax.ShapeDtypeStruct((n, 64), _f32),
            jax.ShapeDtypeStruct((n, 20), _f32),
            jax.ShapeDtypeStruct((n, 128), _f32),
            jax.ShapeDtypeStruct((n, 512), _f32),
        ],
    )(x, *consts)


def _spmm_body(flags, *refs):
    (act, has_mix, has_next, want_t32, want_tbf, want_u32, want_ubf) = flags
    it = iter(refs)
    adj_ref = next(it)
    y_ref = next(it)
    mix_ref = next(it) if has_mix else None
    w_ref = next(it) if has_next else None
    acc = jnp.dot(adj_ref[...], y_ref[...].astype(_bf16),
                  preferred_element_type=_f32)
    t = jnp.tanh(acc) if act else acc
    u = (1.0 - _SIGMA) * mix_ref[...] + _SIGMA * t if has_mix else t
    if want_t32:
        next(it)[...] = t
    if want_tbf:
        next(it)[...] = t.astype(_bf16)
    if want_u32:
        next(it)[...] = u
    if want_ubf:
        next(it)[...] = u.astype(_bf16)
    if has_next:
        next(it)[...] = _dot(u, w_ref[...])


def _spmm_stage(adj_bf, y, mix=None, w_next=None, act=True,
                want_t32=False, want_tbf=False, want_u32=False,
                want_ubf=False):
    n = adj_bf.shape[0]
    kin = y.shape[1]
    grid = (n // _BM,)
    row = lambda k: pl.BlockSpec((_BM, k), lambda i: (i, 0))
    in_specs = [pl.BlockSpec((_BM, n), lambda i: (i, 0)), _full(y)]
    operands = [adj_bf, y]
    if mix is not None:
        in_specs.append(row(kin))
        operands.append(mix)
    if w_next is not None:
        in_specs.append(_full(w_next))
        operands.append(w_next)
    out_specs, out_shape = [], []
    if want_t32:
        out_specs.append(row(kin))
        out_shape.append(jax.ShapeDtypeStruct((n, kin), _f32))
    if want_tbf:
        out_specs.append(row(kin))
        out_shape.append(jax.ShapeDtypeStruct((n, kin), _bf16))
    if want_u32:
        out_specs.append(row(kin))
        out_shape.append(jax.ShapeDtypeStruct((n, kin), _f32))
    if want_ubf:
        out_specs.append(row(kin))
        out_shape.append(jax.ShapeDtypeStruct((n, kin), _bf16))
    if w_next is not None:
        kout = w_next.shape[1]
        out_specs.append(row(kout))
        out_shape.append(jax.ShapeDtypeStruct((n, kout), _f32))
    flags = (act, mix is not None, w_next is not None,
             want_t32, want_tbf, want_u32, want_ubf)
    outs = pl.pallas_call(
        functools.partial(_spmm_body, flags),
        grid=grid,
        in_specs=in_specs,
        out_specs=out_specs,
        out_shape=out_shape,
    )(*operands)
    return outs


def _sim_body(zt_ref, zh_ref, out_ref):
    i = pl.program_id(0)
    j = pl.program_id(1)
    dims = (((1,), (1,)), ((), ()))
    zt_i = zt_ref[pl.ds(i * _BS, _BS), :]
    zt_j = zt_ref[pl.ds(j * _BS, _BS), :]
    zh_i = zh_ref[pl.ds(i * _BS, _BS), :]
    zh_j = zh_ref[pl.ds(j * _BS, _BS), :]
    l1 = jax.lax.dot_general(zt_i, zt_j, dims, preferred_element_type=_f32)
    l2 = jax.lax.dot_general(zh_i, zh_j, dims, preferred_element_type=_f32)
    out_ref[...] = jax.nn.sigmoid(l1) + jax.nn.sigmoid(l2)


def _similarity(zt_bf, zh_bf):
    n = zt_bf.shape[0]
    g = n // _BS
    return pl.pallas_call(
        _sim_body,
        grid=(g, g),
        in_specs=[_full2(zt_bf), _full2(zh_bf)],
        out_specs=pl.BlockSpec((_BS, _BS), lambda i, j: (i, j)),
        out_shape=jax.ShapeDtypeStruct((n, n), _f32),
    )(zt_bf, zh_bf)


def _full2(a):
    return pl.BlockSpec(a.shape, lambda i, j: (0, 0))


def kernel(X_tilde1, adj1, W_enc1, b_enc1, W_enc2, b_enc2, W_enc3, b_enc3,
           W_dec1, b_dec1, W_dec2, b_dec2, W_dec3, b_dec3, W_xbar, b_xbar,
           Wg1, Wg2, Wg3, Wg4, Wg5, Wg6):
    r = lambda b: b.reshape(1, -1)
    adj_bf = adj1.astype(_bf16)

    z_ae1, z_ae2, z_ae3, y1, x_hat = _ae_chain(
        X_tilde1, W_enc1, r(b_enc1), W_enc2, r(b_enc2), W_enc3, r(b_enc3),
        W_dec1, r(b_dec1), W_dec2, r(b_dec2), W_dec3, r(b_dec3),
        W_xbar, r(b_xbar), Wg1)

    (y2,) = _spmm_stage(adj_bf, y1, mix=z_ae1, w_next=Wg2, act=True)
    (y3,) = _spmm_stage(adj_bf, y2, mix=z_ae2, w_next=Wg3, act=True)
    z_igae3, z_tilde, zt_bf, y4 = _spmm_stage(
        adj_bf, y3, mix=z_ae3, w_next=Wg4, act=False,
        want_t32=True, want_u32=True, want_ubf=True)
    (y5,) = _spmm_stage(adj_bf, y4, w_next=Wg5, act=True)
    (y6,) = _spmm_stage(adj_bf, y5, w_next=Wg6, act=True)
    z_hat, zh_bf = _spmm_stage(adj_bf, y6, act=True,
                               want_t32=True, want_tbf=True)

    adj_hat = _similarity(zt_bf, zh_bf)
    return (x_hat, z_hat, adj_hat, z_ae3, z_igae3, z_tilde)


# speedup vs baseline: 1.1141x; 1.1109x over previous
import functools

import jax
import jax.numpy as jnp
from jax.experimental import pallas as pl

_N = 4096
_SIGMA = 0.5
_BM = 256
_BS = 512

_f32 = jnp.float32
_bf16 = jnp.bfloat16


def _dot(a, b):
    return jnp.dot(a.astype(_bf16), b.astype(_bf16),
                   preferred_element_type=_f32)


def _k1_body(x_ref, we1, be1, we2, be2, we3, be3,
             wd1, bd1, wd2, bd2, wd3, bd3, wxb, bxb, wg1,
             z1_ref, z2_ref, z3_ref, y1_ref, xhat_ref):
    x = x_ref[...]
    z1 = jax.nn.relu(_dot(x, we1[...]) + be1[...])
    z2 = jax.nn.relu(_dot(z1, we2[...]) + be2[...])
    z3 = _dot(z2, we3[...]) + be3[...]
    z1_ref[...] = z1
    z2_ref[...] = z2
    z3_ref[...] = z3
    y1_ref[...] = _dot(x, wg1[...])
    d = jax.nn.relu(_dot(z3, wd1[...]) + bd1[...])
    d = jax.nn.relu(_dot(d, wd2[...]) + bd2[...])
    d = jax.nn.relu(_dot(d, wd3[...]) + bd3[...])
    xhat_ref[...] = _dot(d, wxb[...]) + bxb[...]


def _full(a):
    nd = a.ndim
    return pl.BlockSpec(a.shape, lambda i: (0,) * nd)


def _ae_chain(x, we1, be1, we2, be2, we3, be3,
              wd1, bd1, wd2, bd2, wd3, bd3, wxb, bxb, wg1_bf):
    n = x.shape[0]
    grid = (n // _BM,)
    row = lambda k: pl.BlockSpec((_BM, k), lambda i: (i, 0))
    consts = [we1, be1, we2, be2, we3, be3,
              wd1, bd1, wd2, bd2, wd3, bd3, wxb, bxb, wg1_bf]
    return pl.pallas_call(
        _k1_body,
        grid=grid,
        in_specs=[row(512)] + [_full(c) for c in consts],
        out_specs=[row(128), row(64), row(20), row(128), row(512)],
        out_shape=[
            jax.ShapeDtypeStruct((n, 128), _f32),
            jax.ShapeDtypeStruct((n, 64), _f32),
            jax.ShapeDtypeStruct((n, 20), _f32),
            jax.ShapeDtypeStruct((n, 128), _f32),
            jax.ShapeDtypeStruct((n, 512), _f32),
        ],
    )(x, *consts)


def _spmm_body(flags, *refs):
    (act, has_mix, has_next, want_t32, want_tbf, want_u32, want_ubf,
     cast_out) = flags
    it = iter(refs)
    adj_ref = next(it)
    y_ref = next(it)
    mix_ref = next(it) if has_mix else None
    w_ref = next(it) if has_next else None
    adj_blk = adj_ref[...].astype(_bf16) if cast_out else adj_ref[...]
    acc = jnp.dot(adj_blk, y_ref[...].astype(_bf16),
                  preferred_element_type=_f32)
    t = jnp.tanh(acc) if act else acc
    u = (1.0 - _SIGMA) * mix_ref[...] + _SIGMA * t if has_mix else t
    if want_t32:
        next(it)[...] = t
    if want_tbf:
        next(it)[...] = t.astype(_bf16)
    if want_u32:
        next(it)[...] = u
    if want_ubf:
        next(it)[...] = u.astype(_bf16)
    if has_next:
        next(it)[...] = _dot(u, w_ref[...])
    if cast_out:
        next(it)[...] = adj_blk


def _spmm_stage(adj_bf, y, mix=None, w_next=None, act=True,
                want_t32=False, want_tbf=False, want_u32=False,
                want_ubf=False, cast_out=False):
    n = adj_bf.shape[0]
    kin = y.shape[1]
    grid = (n // _BM,)
    row = lambda k: pl.BlockSpec((_BM, k), lambda i: (i, 0))
    in_specs = [pl.BlockSpec((_BM, n), lambda i: (i, 0)), _full(y)]
    operands = [adj_bf, y]
    if mix is not None:
        in_specs.append(row(kin))
        operands.append(mix)
    if w_next is not None:
        in_specs.append(_full(w_next))
        operands.append(w_next)
    out_specs, out_shape = [], []
    if want_t32:
        out_specs.append(row(kin))
        out_shape.append(jax.ShapeDtypeStruct((n, kin), _f32))
    if want_tbf:
        out_specs.append(row(kin))
        out_shape.append(jax.ShapeDtypeStruct((n, kin), _bf16))
    if want_u32:
        out_specs.append(row(kin))
        out_shape.append(jax.ShapeDtypeStruct((n, kin), _f32))
    if want_ubf:
        out_specs.append(row(kin))
        out_shape.append(jax.ShapeDtypeStruct((n, kin), _bf16))
    if w_next is not None:
        kout = w_next.shape[1]
        out_specs.append(row(kout))
        out_shape.append(jax.ShapeDtypeStruct((n, kout), _f32))
    if cast_out:
        out_specs.append(pl.BlockSpec((_BM, n), lambda i: (i, 0)))
        out_shape.append(jax.ShapeDtypeStruct((n, n), _bf16))
    flags = (act, mix is not None, w_next is not None,
             want_t32, want_tbf, want_u32, want_ubf, cast_out)
    outs = pl.pallas_call(
        functools.partial(_spmm_body, flags),
        grid=grid,
        in_specs=in_specs,
        out_specs=out_specs,
        out_shape=out_shape,
    )(*operands)
    return outs


def _sim_body(zt_ref, zh_ref, out_ref):
    i = pl.program_id(0)
    j = pl.program_id(1)
    dims = (((1,), (1,)), ((), ()))
    zt_i = zt_ref[pl.ds(i * _BS, _BS), :]
    zt_j = zt_ref[pl.ds(j * _BS, _BS), :]
    zh_i = zh_ref[pl.ds(i * _BS, _BS), :]
    zh_j = zh_ref[pl.ds(j * _BS, _BS), :]
    l1 = jax.lax.dot_general(zt_i, zt_j, dims, preferred_element_type=_f32)
    l2 = jax.lax.dot_general(zh_i, zh_j, dims, preferred_element_type=_f32)
    out_ref[...] = 1.0 + 0.5 * (jnp.tanh(0.5 * l1) + jnp.tanh(0.5 * l2))


def _similarity(zt_bf, zh_bf):
    n = zt_bf.shape[0]
    g = n // _BS
    return pl.pallas_call(
        _sim_body,
        grid=(g, g),
        in_specs=[_full2(zt_bf), _full2(zh_bf)],
        out_specs=pl.BlockSpec((_BS, _BS), lambda i, j: (i, j)),
        out_shape=jax.ShapeDtypeStruct((n, n), _f32),
    )(zt_bf, zh_bf)


def _full2(a):
    return pl.BlockSpec(a.shape, lambda i, j: (0, 0))


def kernel(X_tilde1, adj1, W_enc1, b_enc1, W_enc2, b_enc2, W_enc3, b_enc3,
           W_dec1, b_dec1, W_dec2, b_dec2, W_dec3, b_dec3, W_xbar, b_xbar,
           Wg1, Wg2, Wg3, Wg4, Wg5, Wg6):
    r = lambda b: b.reshape(1, -1)

    z_ae1, z_ae2, z_ae3, y1, x_hat = _ae_chain(
        X_tilde1, W_enc1, r(b_enc1), W_enc2, r(b_enc2), W_enc3, r(b_enc3),
        W_dec1, r(b_dec1), W_dec2, r(b_dec2), W_dec3, r(b_dec3),
        W_xbar, r(b_xbar), Wg1)

    y2, adj_bf = _spmm_stage(adj1, y1, mix=z_ae1, w_next=Wg2, act=True,
                             cast_out=True)
    (y3,) = _spmm_stage(adj_bf, y2, mix=z_ae2, w_next=Wg3, act=True)
    z_igae3, z_tilde, zt_bf, y4 = _spmm_stage(
        adj_bf, y3, mix=z_ae3, w_next=Wg4, act=False,
        want_t32=True, want_u32=True, want_ubf=True)
    (y5,) = _spmm_stage(adj_bf, y4, w_next=Wg5, act=True)
    (y6,) = _spmm_stage(adj_bf, y5, w_next=Wg6, act=True)
    z_hat, zh_bf = _spmm_stage(adj_bf, y6, act=True,
                               want_t32=True, want_tbf=True)

    adj_hat = _similarity(zt_bf, zh_bf)
    return (x_hat, z_hat, adj_hat, z_ae3, z_igae3, z_tilde)
